# baseline (speedup 1.0000x reference)
import functools

import jax
import jax.numpy as jnp
from jax.experimental import pallas as pl
from jax.experimental.pallas import tpu as pltpu

_ALPHA = 0.2
_LOG2E = 1.4426950408889634


def _divisor_block(n, target):
    best = 8
    for b in range(8, target + 1, 8):
        if n % b == 0:
            best = b
    return best


def _proj_a_body(n, br, x_ref, w0_ref, a0_ref, w1_ref, a1_ref,
                 wh0_ref, f10_ref, f20_ref, wh1_ref, f11_ref, f21_ref):
    i = pl.program_id(0)
    rows = jax.lax.broadcasted_iota(jnp.int32, (br, 1), 0) + i * br
    valid = rows < n
    x = jnp.where(valid, x_ref[...], 0.0)
    neg_inf = jnp.float32(-jnp.inf)
    for w_ref, a_ref, wh_ref, f1_ref, f2_ref in (
        (w0_ref, a0_ref, wh0_ref, f10_ref, f20_ref),
        (w1_ref, a1_ref, wh1_ref, f11_ref, f21_ref),
    ):
        d = w_ref.shape[1]
        wh = jnp.dot(x, w_ref[...], preferred_element_type=jnp.float32)
        av = a_ref[...] * _LOG2E
        wh_ref[...] = wh
        f1_ref[...] = jnp.dot(wh, av[0:d, :], preferred_element_type=jnp.float32)
        f2 = jnp.dot(wh, av[d:2 * d, :], preferred_element_type=jnp.float32)
        f2_ref[...] = jnp.where(valid, f2, neg_inf)


def _proj_b_body(n, br, x0_ref, x1_ref, w_ref, a_ref, wh_ref, f1_ref, f2_ref):
    i = pl.program_id(0)
    rows = jax.lax.broadcasted_iota(jnp.int32, (br, 1), 0) + i * br
    valid = rows < n
    x0 = jnp.where(valid, x0_ref[...], 0.0)
    x1 = jnp.where(valid, x1_ref[...], 0.0)
    dh = x0_ref.shape[1]
    d = w_ref.shape[1]
    wh = (jnp.dot(x0, w_ref[0:dh, :], preferred_element_type=jnp.float32)
          + jnp.dot(x1, w_ref[dh:2 * dh, :], preferred_element_type=jnp.float32))
    av = a_ref[...] * _LOG2E
    wh_ref[...] = wh
    f1_ref[...] = jnp.dot(wh, av[0:d, :], preferred_element_type=jnp.float32)
    f2 = jnp.dot(wh, av[d:2 * d, :], preferred_element_type=jnp.float32)
    f2_ref[...] = jnp.where(valid, f2, neg_inf := jnp.float32(-jnp.inf))


def _flash_body(nh, n_valid, apply_elu, *refs):
    adj_ref = refs[0]
    f1_refs = refs[1:1 + nh]
    f2t_refs = refs[1 + nh:1 + 2 * nh]
    wh_refs = refs[1 + 2 * nh:1 + 3 * nh]
    o_refs = refs[1 + 3 * nh:1 + 4 * nh]
    l_refs = refs[1 + 4 * nh:1 + 5 * nh]
    acc_refs = refs[1 + 5 * nh:1 + 6 * nh]
    swh_refs = refs[1 + 6 * nh:1 + 7 * nh]

    j = pl.program_id(1)
    nj = pl.num_programs(1)

    @pl.when(j == 0)
    def _():
        for h in range(nh):
            l_refs[h][...] = jnp.zeros(l_refs[h].shape, jnp.float32)
            acc_refs[h][...] = jnp.zeros(acc_refs[h].shape, jnp.float32)
            swh_refs[h][...] = jnp.zeros(swh_refs[h].shape, jnp.float32)

    adjn = jnp.where(adj_ref[...] > 0.0, 1.0, 0.0)
    for h in range(nh):
        wh = wh_refs[h][...]
        s = f1_refs[h][...] + f2t_refs[h][...]
        s = jnp.maximum(s, _ALPHA * s)
        p = adjn * jnp.exp2(s)
        l_refs[h][...] += jnp.sum(p, axis=1, keepdims=True)
        acc_refs[h][...] += jnp.dot(p, wh,
                                    preferred_element_type=jnp.float32)
        swh_refs[h][...] += jnp.sum(wh, axis=0, keepdims=True)

    @pl.when(j == nj - 1)
    def _():
        inv_n = jnp.float32(1.0 / n_valid)
        for h in range(nh):
            l = l_refs[h][...]
            fb = swh_refs[h][...] * inv_n
            out = jnp.where(l > 0.0, acc_refs[h][...] / l, fb)
            if apply_elu:
                out = jnp.where(out > 0.0, out, jnp.exp(out) - 1.0)
            o_refs[h][...] = out


def _flash_call(adj, f1s, f2ts, whs, apply_elu):
    n = adj.shape[0]
    nh = len(whs)
    d = whs[0].shape[1]
    npad = whs[0].shape[0]
    bi = _divisor_block(n, 1000)
    bj = min(2560, npad)
    gi = n // bi
    gj = npad // bj
    body = functools.partial(_flash_body, nh, n, apply_elu)
    in_specs = [pl.BlockSpec((bi, bj), lambda i, j: (i, j))]
    in_specs += [pl.BlockSpec((bi, 1), lambda i, j: (i, 0))] * nh
    in_specs += [pl.BlockSpec((1, bj), lambda i, j: (0, j))] * nh
    in_specs += [pl.BlockSpec((bj, d), lambda i, j: (j, 0))] * nh
    out_specs = [pl.BlockSpec((bi, d), lambda i, j: (i, 0))] * nh
    out_shape = [jax.ShapeDtypeStruct((n, d), jnp.float32)] * nh
    scratch = ([pltpu.VMEM((bi, 1), jnp.float32)] * nh
               + [pltpu.VMEM((bi, d), jnp.float32)] * nh
               + [pltpu.VMEM((1, d), jnp.float32)] * nh)
    outs = pl.pallas_call(
        body,
        grid=(gi, gj),
        in_specs=in_specs,
        out_specs=out_specs,
        out_shape=out_shape,
        scratch_shapes=scratch,
        compiler_params=pltpu.CompilerParams(
            dimension_semantics=("parallel", "arbitrary")),
    )(adj, *f1s, *f2ts, *whs)
    return outs


def kernel(adjacency, X, W0, a0, W1, a1, W_out, a_out):
    n = adjacency.shape[0]
    d_in = X.shape[1]
    d_hid = W0.shape[1]
    d_out = W_out.shape[1]
    bj = min(2560, n)
    npad = -(-n // bj) * bj
    br = 512
    gr = -(-npad // br)

    wh0, f10, f20, wh1, f11, f21 = pl.pallas_call(
        functools.partial(_proj_a_body, n, br),
        grid=(gr,),
        in_specs=[
            pl.BlockSpec((br, d_in), lambda i: (i, 0)),
            pl.BlockSpec((d_in, d_hid), lambda i: (0, 0)),
            pl.BlockSpec((2 * d_hid, 1), lambda i: (0, 0)),
            pl.BlockSpec((d_in, d_hid), lambda i: (0, 0)),
            pl.BlockSpec((2 * d_hid, 1), lambda i: (0, 0)),
        ],
        out_specs=[
            pl.BlockSpec((br, d_hid), lambda i: (i, 0)),
            pl.BlockSpec((br, 1), lambda i: (i, 0)),
            pl.BlockSpec((br, 1), lambda i: (i, 0)),
            pl.BlockSpec((br, d_hid), lambda i: (i, 0)),
            pl.BlockSpec((br, 1), lambda i: (i, 0)),
            pl.BlockSpec((br, 1), lambda i: (i, 0)),
        ],
        out_shape=[
            jax.ShapeDtypeStruct((npad, d_hid), jnp.float32),
            jax.ShapeDtypeStruct((npad, 1), jnp.float32),
            jax.ShapeDtypeStruct((npad, 1), jnp.float32),
            jax.ShapeDtypeStruct((npad, d_hid), jnp.float32),
            jax.ShapeDtypeStruct((npad, 1), jnp.float32),
            jax.ShapeDtypeStruct((npad, 1), jnp.float32),
        ],
        compiler_params=pltpu.CompilerParams(
            dimension_semantics=("parallel",)),
    )(X, W0, a0, W1, a1)

    h0, h1 = _flash_call(
        adjacency,
        (f10, f11),
        (f20.reshape(1, npad), f21.reshape(1, npad)),
        (wh0, wh1),
        apply_elu=True,
    )

    who_out, f1o, f2o = pl.pallas_call(
        functools.partial(_proj_b_body, n, br),
        grid=(gr,),
        in_specs=[
            pl.BlockSpec((br, d_hid), lambda i: (i, 0)),
            pl.BlockSpec((br, d_hid), lambda i: (i, 0)),
            pl.BlockSpec((2 * d_hid, d_out), lambda i: (0, 0)),
            pl.BlockSpec((2 * d_out, 1), lambda i: (0, 0)),
        ],
        out_specs=[
            pl.BlockSpec((br, d_out), lambda i: (i, 0)),
            pl.BlockSpec((br, 1), lambda i: (i, 0)),
            pl.BlockSpec((br, 1), lambda i: (i, 0)),
        ],
        out_shape=[
            jax.ShapeDtypeStruct((npad, d_out), jnp.float32),
            jax.ShapeDtypeStruct((npad, 1), jnp.float32),
            jax.ShapeDtypeStruct((npad, 1), jnp.float32),
        ],
        compiler_params=pltpu.CompilerParams(
            dimension_semantics=("parallel",)),
    )(h0, h1, W_out, a_out)

    (out,) = _flash_call(
        adjacency,
        (f1o,),
        (f2o.reshape(1, npad),),
        (who_out,),
        apply_elu=False,
    )
    return out


# speedup vs baseline: 1.2389x; 1.2389x over previous
import functools

import jax
import jax.numpy as jnp
from jax.experimental import pallas as pl
from jax.experimental.pallas import tpu as pltpu

_ALPHA = 0.2
_LOG2E = 1.4426950408889634


def _divisor_block(n, target):
    best = 8
    for b in range(8, target + 1, 8):
        if n % b == 0:
            best = b
    return best


def _proj_a_body(n, br, x_ref, w0_ref, a0_ref, w1_ref, a1_ref,
                 wh0_ref, f10_ref, f20_ref, wh1_ref, f11_ref, f21_ref):
    i = pl.program_id(0)
    rows = jax.lax.broadcasted_iota(jnp.int32, (br, 1), 0) + i * br
    valid = rows < n
    x = jnp.where(valid, x_ref[...], 0.0)
    neg_inf = jnp.float32(-jnp.inf)
    for w_ref, a_ref, wh_ref, f1_ref, f2_ref in (
        (w0_ref, a0_ref, wh0_ref, f10_ref, f20_ref),
        (w1_ref, a1_ref, wh1_ref, f11_ref, f21_ref),
    ):
        d = w_ref.shape[1]
        wh = jnp.dot(x, w_ref[...], preferred_element_type=jnp.float32)
        av = a_ref[...] * _LOG2E
        wh_ref[...] = wh
        f1_ref[...] = jnp.dot(wh, av[0:d, :], preferred_element_type=jnp.float32)
        f2 = jnp.dot(wh, av[d:2 * d, :], preferred_element_type=jnp.float32)
        f2_ref[...] = jnp.where(valid, f2, neg_inf)


def _proj_b_body(n, br, x0_ref, x1_ref, w_ref, a_ref, wh_ref, f1_ref, f2_ref):
    i = pl.program_id(0)
    rows = jax.lax.broadcasted_iota(jnp.int32, (br, 1), 0) + i * br
    valid = rows < n
    x0 = jnp.where(valid, x0_ref[...], 0.0)
    x1 = jnp.where(valid, x1_ref[...], 0.0)
    dh = x0_ref.shape[1]
    d = w_ref.shape[1]
    wh = (jnp.dot(x0, w_ref[0:dh, :], preferred_element_type=jnp.float32)
          + jnp.dot(x1, w_ref[dh:2 * dh, :], preferred_element_type=jnp.float32))
    av = a_ref[...] * _LOG2E
    wh_ref[...] = wh
    f1_ref[...] = jnp.dot(wh, av[0:d, :], preferred_element_type=jnp.float32)
    f2 = jnp.dot(wh, av[d:2 * d, :], preferred_element_type=jnp.float32)
    f2_ref[...] = jnp.where(valid, f2, neg_inf := jnp.float32(-jnp.inf))


def _flash_body(nh, d_out, n_valid, apply_elu, *refs):
    adj_ref = refs[0]
    f1_refs = refs[1:1 + nh]
    f2t_refs = refs[1 + nh:1 + 2 * nh]
    wh_refs = refs[1 + 2 * nh:1 + 3 * nh]
    o_refs = refs[1 + 3 * nh:1 + 4 * nh]
    acc_refs = refs[1 + 4 * nh:1 + 5 * nh]
    swh_refs = refs[1 + 5 * nh:1 + 6 * nh]

    j = pl.program_id(1)
    nj = pl.num_programs(1)

    @pl.when(j == 0)
    def _():
        for h in range(nh):
            acc_refs[h][...] = jnp.zeros(acc_refs[h].shape, jnp.float32)
            swh_refs[h][...] = jnp.zeros(swh_refs[h].shape, jnp.float32)

    adjn = jnp.where(adj_ref[...] > 0.0, 1.0, 0.0)
    for h in range(nh):
        wh = wh_refs[h][...]
        s = f1_refs[h][...] + f2t_refs[h][...]
        s = jnp.maximum(s, _ALPHA * s)
        p = adjn * jnp.exp2(s)
        acc_refs[h][...] += jnp.dot(p, wh,
                                    preferred_element_type=jnp.float32)
        swh_refs[h][...] += jnp.sum(wh, axis=0, keepdims=True)

    @pl.when(j == nj - 1)
    def _():
        inv_n = jnp.float32(1.0 / n_valid)
        for h in range(nh):
            a = acc_refs[h][...]
            l = a[:, d_out:d_out + 1]
            fb = swh_refs[h][:, 0:d_out] * inv_n
            out = jnp.where(l > 0.0, a[:, 0:d_out] / l, fb)
            if apply_elu:
                out = jnp.where(out > 0.0, out, jnp.exp(out) - 1.0)
            o_refs[h][...] = out


def _flash_call(adj, f1s, f2ts, whs, d_out, apply_elu):
    n = adj.shape[0]
    nh = len(whs)
    da = whs[0].shape[1]
    npad = whs[0].shape[0]
    bi = _divisor_block(n, 1000)
    bj = min(2560, npad)
    gi = n // bi
    gj = npad // bj
    body = functools.partial(_flash_body, nh, d_out, n, apply_elu)
    in_specs = [pl.BlockSpec((bi, bj), lambda i, j: (i, j))]
    in_specs += [pl.BlockSpec((bi, 1), lambda i, j: (i, 0))] * nh
    in_specs += [pl.BlockSpec((1, bj), lambda i, j: (0, j))] * nh
    in_specs += [pl.BlockSpec((bj, da), lambda i, j: (j, 0))] * nh
    out_specs = [pl.BlockSpec((bi, d_out), lambda i, j: (i, 0))] * nh
    out_shape = [jax.ShapeDtypeStruct((n, d_out), jnp.float32)] * nh
    scratch = ([pltpu.VMEM((bi, da), jnp.float32)] * nh
               + [pltpu.VMEM((1, da), jnp.float32)] * nh)
    outs = pl.pallas_call(
        body,
        grid=(gi, gj),
        in_specs=in_specs,
        out_specs=out_specs,
        out_shape=out_shape,
        scratch_shapes=scratch,
        compiler_params=pltpu.CompilerParams(
            dimension_semantics=("parallel", "arbitrary")),
    )(adj, *f1s, *f2ts, *whs)
    return outs


def kernel(adjacency, X, W0, a0, W1, a1, W_out, a_out):
    n = adjacency.shape[0]
    d_in = X.shape[1]
    d_hid = W0.shape[1]
    d_out = W_out.shape[1]
    bj = min(2560, n)
    npad = -(-n // bj) * bj
    br = 512
    gr = -(-npad // br)

    wh0, f10, f20, wh1, f11, f21 = pl.pallas_call(
        functools.partial(_proj_a_body, n, br),
        grid=(gr,),
        in_specs=[
            pl.BlockSpec((br, d_in), lambda i: (i, 0)),
            pl.BlockSpec((d_in, d_hid), lambda i: (0, 0)),
            pl.BlockSpec((2 * d_hid, 1), lambda i: (0, 0)),
            pl.BlockSpec((d_in, d_hid), lambda i: (0, 0)),
            pl.BlockSpec((2 * d_hid, 1), lambda i: (0, 0)),
        ],
        out_specs=[
            pl.BlockSpec((br, d_hid), lambda i: (i, 0)),
            pl.BlockSpec((br, 1), lambda i: (i, 0)),
            pl.BlockSpec((br, 1), lambda i: (i, 0)),
            pl.BlockSpec((br, d_hid), lambda i: (i, 0)),
            pl.BlockSpec((br, 1), lambda i: (i, 0)),
            pl.BlockSpec((br, 1), lambda i: (i, 0)),
        ],
        out_shape=[
            jax.ShapeDtypeStruct((npad, d_hid), jnp.float32),
            jax.ShapeDtypeStruct((npad, 1), jnp.float32),
            jax.ShapeDtypeStruct((npad, 1), jnp.float32),
            jax.ShapeDtypeStruct((npad, d_hid), jnp.float32),
            jax.ShapeDtypeStruct((npad, 1), jnp.float32),
            jax.ShapeDtypeStruct((npad, 1), jnp.float32),
        ],
        compiler_params=pltpu.CompilerParams(
            dimension_semantics=("parallel",)),
    )(X, W0, a0, W1, a1)

    ones_col = (jnp.arange(npad) < n).astype(jnp.float32).reshape(npad, 1)
    zpad = jnp.zeros((npad, d_hid - 1), jnp.float32)
    h0, h1 = _flash_call(
        adjacency,
        (f10, f11),
        (f20.reshape(1, npad), f21.reshape(1, npad)),
        (jnp.concatenate([wh0, ones_col, zpad], axis=1),
         jnp.concatenate([wh1, ones_col, zpad], axis=1)),
        d_out=d_hid,
        apply_elu=True,
    )

    who_out, f1o, f2o = pl.pallas_call(
        functools.partial(_proj_b_body, n, br),
        grid=(gr,),
        in_specs=[
            pl.BlockSpec((br, d_hid), lambda i: (i, 0)),
            pl.BlockSpec((br, d_hid), lambda i: (i, 0)),
            pl.BlockSpec((2 * d_hid, d_out), lambda i: (0, 0)),
            pl.BlockSpec((2 * d_out, 1), lambda i: (0, 0)),
        ],
        out_specs=[
            pl.BlockSpec((br, d_out), lambda i: (i, 0)),
            pl.BlockSpec((br, 1), lambda i: (i, 0)),
            pl.BlockSpec((br, 1), lambda i: (i, 0)),
        ],
        out_shape=[
            jax.ShapeDtypeStruct((npad, d_out), jnp.float32),
            jax.ShapeDtypeStruct((npad, 1), jnp.float32),
            jax.ShapeDtypeStruct((npad, 1), jnp.float32),
        ],
        compiler_params=pltpu.CompilerParams(
            dimension_semantics=("parallel",)),
    )(h0, h1, W_out, a_out)

    zpad_o = jnp.zeros((npad, d_out - 1), jnp.float32)
    (out,) = _flash_call(
        adjacency,
        (f1o,),
        (f2o.reshape(1, npad),),
        (jnp.concatenate([who_out, ones_col, zpad_o], axis=1),),
        d_out=d_out,
        apply_elu=False,
    )
    return out


# speedup vs baseline: 1.2773x; 1.0310x over previous
import functools

import jax
import jax.numpy as jnp
from jax.experimental import pallas as pl
from jax.experimental.pallas import tpu as pltpu

_ALPHA = 0.2
_LOG2E = 1.4426950408889634


def _divisor_block(n, target):
    best = 8
    for b in range(8, target + 1, 8):
        if n % b == 0:
            best = b
    return best


def _proj_a_body(n, br, x_ref, w0_ref, a0_ref, w1_ref, a1_ref,
                 wh0_ref, f10_ref, f20_ref, wh1_ref, f11_ref, f21_ref):
    i = pl.program_id(0)
    rows = jax.lax.broadcasted_iota(jnp.int32, (br, 1), 0) + i * br
    valid = rows < n
    x = jnp.where(valid, x_ref[...], 0.0)
    neg_inf = jnp.float32(-jnp.inf)
    for w_ref, a_ref, wh_ref, f1_ref, f2_ref in (
        (w0_ref, a0_ref, wh0_ref, f10_ref, f20_ref),
        (w1_ref, a1_ref, wh1_ref, f11_ref, f21_ref),
    ):
        d = w_ref.shape[1]
        wh = jnp.dot(x, w_ref[...], preferred_element_type=jnp.float32)
        av = a_ref[...] * _LOG2E
        wh_ref[...] = jnp.concatenate(
            [wh, valid.astype(jnp.float32),
             jnp.zeros((wh.shape[0], d - 1), jnp.float32)], axis=1)
        f1_ref[...] = jnp.dot(wh, av[0:d, :], preferred_element_type=jnp.float32)
        f2 = jnp.dot(wh, av[d:2 * d, :], preferred_element_type=jnp.float32)
        f2_ref[...] = jnp.where(valid, f2, neg_inf)


def _proj_b_body(n, br, x0_ref, x1_ref, w_ref, a_ref, wh_ref, f1_ref, f2_ref):
    i = pl.program_id(0)
    rows = jax.lax.broadcasted_iota(jnp.int32, (br, 1), 0) + i * br
    valid = rows < n
    x0 = jnp.where(valid, x0_ref[...], 0.0)
    x1 = jnp.where(valid, x1_ref[...], 0.0)
    dh = x0_ref.shape[1]
    d = w_ref.shape[1]
    wh = (jnp.dot(x0, w_ref[0:dh, :], preferred_element_type=jnp.float32)
          + jnp.dot(x1, w_ref[dh:2 * dh, :], preferred_element_type=jnp.float32))
    av = a_ref[...] * _LOG2E
    wh_ref[...] = jnp.concatenate(
        [wh, valid.astype(jnp.float32),
         jnp.zeros((wh.shape[0], d - 1), jnp.float32)], axis=1)
    f1_ref[...] = jnp.dot(wh, av[0:d, :], preferred_element_type=jnp.float32)
    f2 = jnp.dot(wh, av[d:2 * d, :], preferred_element_type=jnp.float32)
    f2_ref[...] = jnp.where(valid, f2, neg_inf := jnp.float32(-jnp.inf))


def _flash_body(nh, d_out, n_valid, apply_elu, *refs):
    adj_ref = refs[0]
    f1_refs = refs[1:1 + nh]
    f2t_refs = refs[1 + nh:1 + 2 * nh]
    wh_refs = refs[1 + 2 * nh:1 + 3 * nh]
    o_refs = refs[1 + 3 * nh:1 + 4 * nh]
    acc_refs = refs[1 + 4 * nh:1 + 5 * nh]
    swh_refs = refs[1 + 5 * nh:1 + 6 * nh]

    j = pl.program_id(1)
    nj = pl.num_programs(1)

    @pl.when(j == 0)
    def _():
        for h in range(nh):
            acc_refs[h][...] = jnp.zeros(acc_refs[h].shape, jnp.float32)
            swh_refs[h][...] = jnp.zeros(swh_refs[h].shape, jnp.float32)

    adjn = jnp.where(adj_ref[...] > 0.0, 1.0, 0.0)
    for h in range(nh):
        wh = wh_refs[h][...]
        s = f1_refs[h][...] + f2t_refs[h][...]
        s = jnp.maximum(s, _ALPHA * s)
        p = adjn * jnp.exp2(s)
        acc_refs[h][...] += jnp.dot(p, wh,
                                    preferred_element_type=jnp.float32)
        swh_refs[h][...] += jnp.sum(wh, axis=0, keepdims=True)

    @pl.when(j == nj - 1)
    def _():
        inv_n = jnp.float32(1.0 / n_valid)
        for h in range(nh):
            a = acc_refs[h][...]
            l = a[:, d_out:d_out + 1]
            fb = swh_refs[h][:, 0:d_out] * inv_n
            out = jnp.where(l > 0.0, a[:, 0:d_out] / l, fb)
            if apply_elu:
                out = jnp.where(out > 0.0, out, jnp.exp(out) - 1.0)
            o_refs[h][...] = out


def _flash_call(adj, f1s, f2ts, whs, d_out, apply_elu):
    n = adj.shape[0]
    nh = len(whs)
    da = whs[0].shape[1]
    npad = whs[0].shape[0]
    bi = _divisor_block(n, 1000)
    bj = min(2560, npad)
    gi = n // bi
    gj = npad // bj
    body = functools.partial(_flash_body, nh, d_out, n, apply_elu)
    in_specs = [pl.BlockSpec((bi, bj), lambda i, j: (i, j))]
    in_specs += [pl.BlockSpec((bi, 1), lambda i, j: (i, 0))] * nh
    in_specs += [pl.BlockSpec((1, bj), lambda i, j: (0, j))] * nh
    in_specs += [pl.BlockSpec((bj, da), lambda i, j: (j, 0))] * nh
    out_specs = [pl.BlockSpec((bi, d_out), lambda i, j: (i, 0))] * nh
    out_shape = [jax.ShapeDtypeStruct((n, d_out), jnp.float32)] * nh
    scratch = ([pltpu.VMEM((bi, da), jnp.float32)] * nh
               + [pltpu.VMEM((1, da), jnp.float32)] * nh)
    outs = pl.pallas_call(
        body,
        grid=(gi, gj),
        in_specs=in_specs,
        out_specs=out_specs,
        out_shape=out_shape,
        scratch_shapes=scratch,
        compiler_params=pltpu.CompilerParams(
            dimension_semantics=("parallel", "arbitrary")),
    )(adj, *f1s, *f2ts, *whs)
    return outs


def kernel(adjacency, X, W0, a0, W1, a1, W_out, a_out):
    n = adjacency.shape[0]
    d_in = X.shape[1]
    d_hid = W0.shape[1]
    d_out = W_out.shape[1]
    bj = min(2560, n)
    npad = -(-n // bj) * bj
    br = 512
    gr = -(-npad // br)

    wh0, f10, f20, wh1, f11, f21 = pl.pallas_call(
        functools.partial(_proj_a_body, n, br),
        grid=(gr,),
        in_specs=[
            pl.BlockSpec((br, d_in), lambda i: (i, 0)),
            pl.BlockSpec((d_in, d_hid), lambda i: (0, 0)),
            pl.BlockSpec((2 * d_hid, 1), lambda i: (0, 0)),
            pl.BlockSpec((d_in, d_hid), lambda i: (0, 0)),
            pl.BlockSpec((2 * d_hid, 1), lambda i: (0, 0)),
        ],
        out_specs=[
            pl.BlockSpec((br, 2 * d_hid), lambda i: (i, 0)),
            pl.BlockSpec((br, 1), lambda i: (i, 0)),
            pl.BlockSpec((br, 1), lambda i: (i, 0)),
            pl.BlockSpec((br, 2 * d_hid), lambda i: (i, 0)),
            pl.BlockSpec((br, 1), lambda i: (i, 0)),
            pl.BlockSpec((br, 1), lambda i: (i, 0)),
        ],
        out_shape=[
            jax.ShapeDtypeStruct((npad, 2 * d_hid), jnp.float32),
            jax.ShapeDtypeStruct((npad, 1), jnp.float32),
            jax.ShapeDtypeStruct((npad, 1), jnp.float32),
            jax.ShapeDtypeStruct((npad, 2 * d_hid), jnp.float32),
            jax.ShapeDtypeStruct((npad, 1), jnp.float32),
            jax.ShapeDtypeStruct((npad, 1), jnp.float32),
        ],
        compiler_params=pltpu.CompilerParams(
            dimension_semantics=("parallel",)),
    )(X, W0, a0, W1, a1)

    h0, h1 = _flash_call(
        adjacency,
        (f10, f11),
        (f20.reshape(1, npad), f21.reshape(1, npad)),
        (wh0, wh1),
        d_out=d_hid,
        apply_elu=True,
    )

    who_out, f1o, f2o = pl.pallas_call(
        functools.partial(_proj_b_body, n, br),
        grid=(gr,),
        in_specs=[
            pl.BlockSpec((br, d_hid), lambda i: (i, 0)),
            pl.BlockSpec((br, d_hid), lambda i: (i, 0)),
            pl.BlockSpec((2 * d_hid, d_out), lambda i: (0, 0)),
            pl.BlockSpec((2 * d_out, 1), lambda i: (0, 0)),
        ],
        out_specs=[
            pl.BlockSpec((br, 2 * d_out), lambda i: (i, 0)),
            pl.BlockSpec((br, 1), lambda i: (i, 0)),
            pl.BlockSpec((br, 1), lambda i: (i, 0)),
        ],
        out_shape=[
            jax.ShapeDtypeStruct((npad, 2 * d_out), jnp.float32),
            jax.ShapeDtypeStruct((npad, 1), jnp.float32),
            jax.ShapeDtypeStruct((npad, 1), jnp.float32),
        ],
        compiler_params=pltpu.CompilerParams(
            dimension_semantics=("parallel",)),
    )(h0, h1, W_out, a_out)

    (out,) = _flash_call(
        adjacency,
        (f1o,),
        (f2o.reshape(1, npad),),
        (who_out,),
        d_out=d_out,
        apply_elu=False,
    )
    return out
